# Initial kernel scaffold; baseline (speedup 1.0000x reference)
import jax
import jax.numpy as jnp
from jax.experimental import pallas as pl


def kernel(query, key):
    raise NotImplementedError("write your pallas kernel here")


# speedup vs baseline: 3.2386x; 3.2386x over previous
import functools
import jax
import jax.numpy as jnp
from jax.experimental import pallas as pl
from jax.experimental.pallas import tpu as pltpu

_DIM = 128
_BASE = 10000.0
_SEQ_BLK = 256


@functools.lru_cache(maxsize=None)
def _tables(sq):
    freqs = 1.0 / (_BASE ** (jnp.arange(0, _DIM, 2)[: _DIM // 2].astype(jnp.float32) / _DIM))
    t = jnp.arange(sq).astype(jnp.float32)
    f = jnp.outer(t, freqs)
    cos = jnp.cos(f)
    sin = jnp.sin(f)
    zeros = jnp.zeros_like(sin)
    c_full = jnp.repeat(cos, 2, axis=1)
    a_full = jnp.stack([-sin, zeros], axis=-1).reshape(sq, _DIM)
    b_full = jnp.stack([zeros, sin], axis=-1).reshape(sq, _DIM)
    return (c_full.reshape(sq, 1, _DIM), a_full.reshape(sq, 1, _DIM),
            b_full.reshape(sq, 1, _DIM))


def _rope_body(c_ref, a_ref, b_ref, q_ref, k_ref, qo_ref, ko_ref):
    c = c_ref[...]
    a = a_ref[...]
    b = b_ref[...]
    for x_ref, o_ref in ((q_ref, qo_ref), (k_ref, ko_ref)):
        x = x_ref[...]
        lo = pltpu.roll(x, _DIM - 1, axis=2)
        hi = pltpu.roll(x, 1, axis=2)
        o_ref[...] = x * c + lo * a + hi * b


def kernel(query, key):
    sq, bsz, nh, hh = query.shape
    c_t, a_t, b_t = _tables(sq)
    fl = bsz * nh
    q3 = query.reshape(sq, fl, hh)
    k3 = key.reshape(sq, fl, hh)

    blk = _SEQ_BLK if sq % _SEQ_BLK == 0 else sq
    grid = (sq // blk,)
    tab_spec = pl.BlockSpec((blk, 1, hh), lambda i: (i, 0, 0))
    dat_spec = pl.BlockSpec((blk, fl, hh), lambda i: (i, 0, 0))

    qo, ko = pl.pallas_call(
        _rope_body,
        grid=grid,
        in_specs=[tab_spec, tab_spec, tab_spec, dat_spec, dat_spec],
        out_specs=[dat_spec, dat_spec],
        out_shape=[
            jax.ShapeDtypeStruct((sq, fl, hh), query.dtype),
            jax.ShapeDtypeStruct((sq, fl, hh), key.dtype),
        ],
    )(c_t, a_t, b_t, q3, k3)
    return qo.reshape(query.shape), ko.reshape(key.shape)
